# baseline (speedup 1.0000x reference)
import jax
import jax.numpy as jnp
from jax.experimental import pallas as pl


_GRID = 4


def _body(idx_ref, emb_ref, out_ref):
    i = pl.program_id(0)
    rs = jax.lax.dot_general(
        jnp.ones((1, emb_ref.shape[1]), jnp.float32),
        emb_ref[...],
        (((1,), (1,)), ((), ())),
        preferred_element_type=jnp.float32,
        precision=jax.lax.Precision.HIGHEST,
    )
    idx = idx_ref[...]
    table = jnp.broadcast_to(rs, (idx.shape[0], rs.shape[1]))
    vals = jnp.take_along_axis(table, idx, axis=1)
    part = jnp.sum(vals, keepdims=True).reshape(1, 1)

    @pl.when(i == 0)
    def _():
        out_ref[...] = part - 1.0

    @pl.when(i > 0)
    def _():
        out_ref[...] += part


def kernel(input_batch, emb36a, emb36b, emb48):
    del emb36a, emb36b
    n, c = input_batch.shape
    block = n // _GRID
    out = pl.pallas_call(
        _body,
        grid=(_GRID,),
        in_specs=[
            pl.BlockSpec((block, c), lambda i: (i, 0)),
            pl.BlockSpec(emb48.shape, lambda i: (0, 0)),
        ],
        out_specs=pl.BlockSpec((1, 1), lambda i: (0, 0)),
        out_shape=jax.ShapeDtypeStruct((1, 1), jnp.float32),
    )(input_batch, emb48)
    return out.reshape(())
